# Initial kernel scaffold; baseline (speedup 1.0000x reference)
import jax
import jax.numpy as jnp
from jax.experimental import pallas as pl


def kernel(images):
    raise NotImplementedError("write your pallas kernel here")


# speedup vs baseline: 1.5706x; 1.5706x over previous
import numpy as np
import jax
import jax.numpy as jnp
from jax.experimental import pallas as pl
from jax.experimental.pallas import tpu as pltpu

_P = 24
_R = 3
_NBINS = _P + 2
_H = 384
_W = 384
_PAD = _R + 1

_SROWS = 400
_SCOLS = 512

def _neighbor_offsets():
    offs = []
    for i in range(_P):
        theta = 2.0 * np.pi * i / _P
        rp = float(np.round(-_R * np.sin(theta), 8))
        cp = float(np.round(_R * np.cos(theta), 8))
        minr = int(np.floor(rp))
        minc = int(np.floor(cp))
        tr = rp - minr
        tc = cp - minc
        offs.append((minr, minc, tr, tc))
    return offs

_OFFS = _neighbor_offsets()


def _lbp_kernel(img_ref, out_ref, pad_ref):
    x = jnp.clip(img_ref[0], 0.0, 1.0)
    x = jnp.floor(x * 255.0)
    gray = jnp.round(0.299 * x[0] + 0.587 * x[1] + 0.114 * x[2])

    pad_ref[...] = jnp.zeros((_SROWS, _SCOLS), jnp.float32)
    pad_ref[_PAD:_PAD + _H, _PAD:_PAD + _W] = gray

    def tap(minr, minc):
        r0 = _PAD + minr
        c0 = _PAD + minc
        return pad_ref[r0:r0 + _H, c0:c0 + _W]

    ones = jnp.zeros((_H, _W), jnp.float32)
    changes = jnp.zeros((_H, _W), jnp.float32)
    prev = None
    first = None
    for (minr, minc, tr, tc) in _OFFS:
        terms = []
        w00 = (1.0 - tr) * (1.0 - tc)
        w10 = tr * (1.0 - tc)
        w01 = (1.0 - tr) * tc
        w11 = tr * tc
        if w00 != 0.0:
            terms.append(w00 * tap(minr, minc) if w00 != 1.0 else tap(minr, minc))
        if w10 != 0.0:
            terms.append(w10 * tap(minr + 1, minc))
        if w01 != 0.0:
            terms.append(w01 * tap(minr, minc + 1))
        if w11 != 0.0:
            terms.append(w11 * tap(minr + 1, minc + 1))
        neigh = terms[0]
        for t in terms[1:]:
            neigh = neigh + t
        bit = jnp.where(neigh - gray >= 0.0, 1.0, 0.0)
        ones = ones + bit
        if prev is None:
            first = bit
        else:
            changes = changes + jnp.abs(prev - bit)
        prev = bit
    changes = changes + jnp.abs(prev - first)

    uniform = changes <= 2.0
    counts = []
    for k in range(_NBINS - 1):
        counts.append(jnp.sum(jnp.where(uniform & (ones == float(k)), 1.0, 0.0)))
    counts.append(jnp.sum(jnp.where(uniform, 0.0, 1.0)))

    inv_n = 1.0 / float(_H * _W)
    hvals = [c * inv_n for c in counts]
    total = hvals[0]
    for v in hvals[1:]:
        total = total + v
    denom = total + 1e-7
    hvec = jnp.concatenate(
        [jnp.broadcast_to((v / denom)[None, None], (1, 1)) for v in hvals],
        axis=1)
    out_ref[...] = hvec[None]


def kernel(images):
    B = images.shape[0]
    out = pl.pallas_call(
        _lbp_kernel,
        grid=(B,),
        in_specs=[pl.BlockSpec((1, 3, _H, _W), lambda b: (b, 0, 0, 0))],
        out_specs=pl.BlockSpec((1, 1, _NBINS), lambda b: (b, 0, 0)),
        out_shape=jax.ShapeDtypeStruct((B, 1, _NBINS), jnp.float32),
        scratch_shapes=[pltpu.VMEM((_SROWS, _SCOLS), jnp.float32)],
    )(images)
    return out.reshape(B, _NBINS)
